# baseline (speedup 1.0000x reference)
import jax
import jax.numpy as jnp
from jax import lax
from jax.experimental import pallas as pl
from jax.experimental.pallas import tpu as pltpu

B, HW, C, K = 16, 196, 384, 12
REL = 48
H = W = 14
N = B * HW
PAD = 144
ROWS = PAD + HW + PAD
F32 = jnp.float32
BF16 = jnp.bfloat16


def _dot(a, b, dims):
    return lax.dot_general(a, b, (dims, ((), ())), preferred_element_type=F32)


def _s1_body(x_ref, w0, w01, w02, w11, w12, w21, w22, cb_ref, W3_ref,
             xs_ref, xh_ref, pad_ref, s0_ref, t0_ref, t1_ref):
    b = pl.program_id(0)
    wio = lax.broadcasted_iota(jnp.int32, (HW, 1), 0) % W

    @pl.when(b == 0)
    def _():
        pad_ref[...] = jnp.zeros((ROWS, C), F32)

    def put(img):
        pad_ref[pl.ds(PAD, HW), :] = img

    def conv(wt_ref, kh, kw, ph, pw):
        acc = jnp.zeros((HW, C), F32)
        t = 0
        for i in range(kh):
            for j in range(kw):
                delta = (i - ph) * W + (j - pw)
                sl = pad_ref[pl.ds(PAD + delta, HW), :]
                dw = j - pw
                if dw != 0:
                    m = (wio + dw >= 0) & (wio + dw < W)
                    sl = jnp.where(m, sl, 0.0)
                acc = acc + sl * wt_ref[pl.ds(t, 1), :]
                t += 1
        return acc

    def brow(i):
        return cb_ref[pl.ds(i, 1), :]

    X = x_ref[0]
    put(X)
    s0 = conv(w0, 5, 5, 2, 2) + brow(0)
    s0_ref[...] = s0
    put(s0)
    a0 = jnp.zeros((HW, C), F32)
    a1 = jnp.zeros((HW, C), F32)
    a2 = jnp.zeros((HW, C), F32)
    for dw in range(-10, 11):
        sl = pad_ref[pl.ds(PAD + dw, HW), :]
        m = (wio + dw >= 0) & (wio + dw < W)
        sl = jnp.where(m, sl, 0.0)
        a2 = a2 + sl * w21[pl.ds(dw + 10, 1), :]
        if abs(dw) <= 5:
            a1 = a1 + sl * w11[pl.ds(dw + 5, 1), :]
        if abs(dw) <= 3:
            a0 = a0 + sl * w01[pl.ds(dw + 3, 1), :]
    t0_ref[...] = a0 + brow(1)
    t1_ref[...] = a1 + brow(3)
    aw2 = a2 + brow(5)
    put(aw2)
    acc = s0_ref[...] + conv(w22, 21, 1, 10, 0) + brow(6)
    put(t0_ref[...])
    acc = acc + conv(w02, 7, 1, 3, 0) + brow(2)
    put(t1_ref[...])
    acc = acc + conv(w12, 11, 1, 5, 0) + brow(4)
    attn3 = _dot(acc.astype(BF16), W3_ref[...], ((1,), (1,))) + brow(7)
    xs_ref[0] = attn3 * X
    xh = jnp.zeros((W, C), F32)
    for t in range(H):
        xh = xh + X[t * W:(t + 1) * W, :]
    xh_ref[0] = xh


def _stage1(x, taps, cbias, W3):
    w0, w01, w02, w11, w12, w21, w22 = taps
    full = lambda s: pl.BlockSpec(s, lambda b: tuple(0 for _ in s))
    return pl.pallas_call(
        _s1_body,
        grid=(B,),
        in_specs=[pl.BlockSpec((1, HW, C), lambda b: (b, 0, 0))]
        + [full(w.shape) for w in (w0, w01, w02, w11, w12, w21, w22)]
        + [full(cbias.shape), full(W3.shape)],
        out_specs=[pl.BlockSpec((1, HW, C), lambda b: (b, 0, 0)),
                   pl.BlockSpec((1, W, C), lambda b: (b, 0, 0))],
        out_shape=[jax.ShapeDtypeStruct((B, HW, C), F32),
                   jax.ShapeDtypeStruct((B, W, C), F32)],
        scratch_shapes=[pltpu.VMEM((ROWS, C), F32), pltpu.VMEM((HW, C), F32),
                        pltpu.VMEM((HW, C), F32), pltpu.VMEM((HW, C), F32)],
        compiler_params=pltpu.CompilerParams(
            dimension_semantics=("arbitrary",)),
    )(x, w0, w01, w02, w11, w12, w21, w22, cbias, W3)


def _s2_body(xs_ref, wcat_ref, cb_ref, cg_ref, cbe_ref, out_ref, ab_ref):
    b = pl.program_id(0)

    @pl.when(b == 0)
    def _():
        ssum = jnp.zeros((1, C), F32)
        for bb in range(B):
            ssum = ssum + jnp.sum(xs_ref[bb], axis=0, keepdims=True)
        s = ssum / N
        G = jnp.zeros((C, C), F32)
        for bb in range(B):
            xc = (xs_ref[bb] - s).astype(BF16)
            G = G + _dot(xc, xc, ((0,), (0,)))
        G16 = G.astype(BF16)
        s16 = s.astype(BF16)
        ones = jnp.ones((1, C), F32)
        for k in range(K):
            Ws = wcat_ref[k * C:(k + 1) * C, :]
            Ez = _dot(s16, Ws, ((1,), (1,)))
            WG = _dot(Ws, G16, ((1,), (0,)))
            var = _dot(ones, WG * Ws.astype(F32), ((1,), (1,))) / N
            alpha = cg_ref[:, k * C:(k + 1) * C] / jnp.sqrt(var + 1e-5)
            beta = cbe_ref[:, k * C:(k + 1) * C] - Ez * alpha
            ab_ref[pl.ds(0, 1), k * C:(k + 1) * C] = alpha
            ab_ref[pl.ds(1, 1), k * C:(k + 1) * C] = beta

    Z = _dot(xs_ref[b].astype(BF16), wcat_ref[...], ((1,), (1,)))
    y = jnp.maximum(Z * ab_ref[pl.ds(0, 1), :] + ab_ref[pl.ds(1, 1), :], 0.0)
    ones_n = jnp.ones((1, HW), F32)
    out_ref[...] = _dot(ones_n, y, ((1,), (0,)))[None] / HW


def _stage2(xs, cW, cb, cg, cbe):
    wcat = cW.reshape(K * C, C).astype(BF16)
    cbf = cb.reshape(1, K * C)
    cgf = cg.reshape(1, K * C)
    cbef = cbe.reshape(1, K * C)
    full = lambda s: pl.BlockSpec(s, lambda b: tuple(0 for _ in s))
    out = pl.pallas_call(
        _s2_body,
        grid=(B,),
        in_specs=[full((B, HW, C)), full((K * C, C)), full((1, K * C)),
                  full((1, K * C)), full((1, K * C))],
        out_specs=pl.BlockSpec((1, 1, K * C), lambda b: (b, 0, 0)),
        out_shape=jax.ShapeDtypeStruct((B, 1, K * C), F32),
        scratch_shapes=[pltpu.VMEM((2, K * C), F32)],
        compiler_params=pltpu.CompilerParams(
            dimension_semantics=("arbitrary",)),
    )(xs, wcat, cbf, cgf, cbef)
    return out.reshape(B, K, C)


def _s3_body(v_ref, xh_ref, VW_ref, Vb_ref, UW_ref, Ub_ref, gK_ref, bK_ref,
             c1w_ref, c1b_ref, c2w_ref, c2b_ref, c3w_ref, c3b_ref,
             c4w_ref, c4b_ref, cclW_ref, cclb_ref, cclg_ref, cclbe_ref,
             pW_ref, pb_ref, pg_ref, pbe_ref, sc_ref,
             out_ref, agg_ref, fp_ref):
    ones_row = jnp.ones((1, C), F32)
    pL = pW_ref[:, 0:C]
    pR = pW_ref[:, C:2 * C]

    pool_rows = []
    for b in range(B):
        XHb = xh_ref[b]
        XHm = XHb * (1.0 / W)
        x1 = _dot(XHm, c1w_ref[...], ((1,), (1,))) + c1b_ref[...]
        x2 = _dot(XHm, c2w_ref[...], ((1,), (1,))) + c2b_ref[...]
        TT = jnp.zeros((W, REL), F32)
        for u in range(W):
            TT = TT + jnp.tanh(x1[u:u + 1, :] - x2)
        P1T = _dot(TT, c4w_ref[...], ((1,), (1,))) + float(W) * c4b_ref[...]
        S3T = _dot(XHb, c3w_ref[...], ((1,), (1,))) + float(W) * c3b_ref[...]
        pool_rows.append(jnp.sum(P1T * S3T, axis=0, keepdims=True) / float(HW))
    pool = jnp.concatenate(pool_rows, axis=0)

    y1 = _dot(pool, cclW_ref[...], ((1,), (1,))) + cclb_ref[...]
    m2 = jnp.mean(y1, axis=0, keepdims=True)
    v2 = jnp.mean((y1 - m2) ** 2, axis=0, keepdims=True)
    lin = jnp.maximum(
        (y1 - m2) * lax.rsqrt(v2 + 1e-5) * cclg_ref[...] + cclbe_ref[...], 0.0)

    msum = jnp.zeros((K, 1), F32)
    iot = lax.broadcasted_iota(jnp.int32, (K, K), 1)
    for b in range(B):
        vb = v_ref[b]
        si = _dot(vb, vb, ((1,), (1,)))
        cur = si
        thr = si
        for _ in range(4):
            m = jnp.max(cur, axis=1, keepdims=True)
            thr = m
            eq = cur == m
            idx = jnp.min(jnp.where(eq, iot, 10 ** 6), axis=1, keepdims=True)
            cur = jnp.where(iot == idx, -1e30, cur)
        adj = (si >= thr).astype(F32)
        deg = jnp.sum(adj, axis=1, keepdims=True)
        dinv = lax.rsqrt(deg)
        Vx = _dot(vb, VW_ref[...], ((1,), (1,))) + Vb_ref[...]
        Ux = _dot(vb, UW_ref[...], ((1,), (1,))) + Ub_ref[...]
        aggb = dinv * _dot(adj, dinv * Vx, ((1,), (0,))) + Ux
        agg_ref[b] = aggb
        msum = msum + jnp.sum(aggb, axis=1, keepdims=True)
    mcol = msum / float(B * C)

    vsum = jnp.zeros((K, 1), F32)
    for b in range(B):
        d = agg_ref[b] - mcol
        vsum = vsum + jnp.sum(d * d, axis=1, keepdims=True)
    inv_col = lax.rsqrt(vsum / float(B * C) + 1e-5)

    ms3 = jnp.zeros((1, C), F32)
    for b in range(B):
        aggn = (agg_ref[b] - mcol) * inv_col * gK_ref[...] + bK_ref[...]
        vgnn = jnp.maximum(v_ref[b] + aggn, 0.0)
        fpb = (_dot(vgnn, pL, ((1,), (1,)))
               + _dot(lin[b:b + 1, :], pR, ((1,), (1,)))
               + pb_ref[...])
        fp_ref[b] = fpb
        ms3 = ms3 + jnp.sum(fpb, axis=0, keepdims=True)
    m3 = ms3 / float(B * K)

    vs3 = jnp.zeros((1, C), F32)
    for b in range(B):
        d = fp_ref[b] - m3
        vs3 = vs3 + jnp.sum(d * d, axis=0, keepdims=True)
    inv3 = lax.rsqrt(vs3 / float(B * K) + 1e-5)

    scr = jnp.maximum(sc_ref[...], 0.0)
    sn = jnp.sqrt(jnp.sum(scr * scr, axis=1, keepdims=True))
    scn = scr / jnp.maximum(sn, 1e-12)

    rows = []
    for b in range(B):
        y = (fp_ref[b] - m3) * inv3 * pg_ref[...] + pbe_ref[...]
        f = jnp.maximum(y, 0.0)
        fn = jnp.sqrt(jnp.sum(f * f, axis=1, keepdims=True))
        fhat = f / jnp.maximum(fn, 1e-12)
        rows.append(_dot(ones_row, fhat * scn, ((1,), (1,))))
    out_ref[...] = jnp.concatenate(rows, axis=0)


def _stage3(v_bkc, XH, args):
    full = lambda a: pl.BlockSpec(a.shape, lambda: tuple(0 for _ in a.shape))
    ins = [v_bkc, XH] + list(args)
    return pl.pallas_call(
        _s3_body,
        in_specs=[full(a) for a in ins],
        out_specs=pl.BlockSpec((B, K), lambda: (0, 0)),
        out_shape=jax.ShapeDtypeStruct((B, K), F32),
        scratch_shapes=[pltpu.VMEM((B, K, C), F32), pltpu.VMEM((B, K, C), F32)],
    )(*ins)


def kernel(x, params):
    p = params

    def taps(name, kh, kw):
        w = p[name]["w"]
        return w[:, 0].reshape(C, kh * kw).T

    tap_ws = (taps("conv0", 5, 5), taps("conv0_1", 1, 7), taps("conv0_2", 7, 1),
              taps("conv1_1", 1, 11), taps("conv1_2", 11, 1),
              taps("conv2_1", 1, 21), taps("conv2_2", 21, 1))
    cbias = jnp.stack([p["conv0"]["b"], p["conv0_1"]["b"], p["conv0_2"]["b"],
                       p["conv1_1"]["b"], p["conv1_2"]["b"],
                       p["conv2_1"]["b"], p["conv2_2"]["b"],
                       p["conv3"]["b"]], axis=0)
    W3 = p["conv3"]["w"][:, :, 0, 0].astype(BF16)

    xs, XH = _stage1(x, tap_ws, cbias, W3)

    v_bkc = _stage2(xs, p["cls_W"], p["cls_b"], p["cls_g"], p["cls_be"])

    row = lambda a: a[None, :]
    args = (p["V_W"], row(p["V_b"]), p["U_W"], row(p["U_b"]),
            jnp.broadcast_to(p["bnv_g"][:, None], (K, C)),
            jnp.broadcast_to(p["bnv_b"][:, None], (K, C)),
            p["c1_w"][:, :, 0, 0], row(p["c1_b"]),
            p["c2_w"][:, :, 0, 0], row(p["c2_b"]),
            p["c3_w"][:, :, 0, 0], row(p["c3_b"]),
            p["c4_w"][:, :, 0, 0], row(p["c4_b"]),
            p["ccl_W"], row(p["ccl_b"]), row(p["ccl_g"]), row(p["ccl_be"]),
            p["post_W"], row(p["post_b"]),
            row(p["post_g"]), row(p["post_be"]), p["sc"])
    return _stage3(v_bkc, XH, args)


# speedup vs baseline: 1.2941x; 1.2941x over previous
import jax
import jax.numpy as jnp
from jax import lax
from jax.experimental import pallas as pl
from jax.experimental.pallas import tpu as pltpu

B, HW, C, K = 16, 196, 384, 12
REL = 48
H = W = 14
N = B * HW
PAD = 144
ROWS = PAD + HW + PAD
F32 = jnp.float32
BF16 = jnp.bfloat16


def _dot(a, b, dims):
    return lax.dot_general(a, b, (dims, ((), ())), preferred_element_type=F32)


CC = 128
NCH = C // CC


def _s1_body(x_ref, w0, w01, w02, w11, w12, w21, w22, cb_ref, W3_ref,
             xs_ref, xh_ref, pad_ref, s0_ref, t0_ref, t1_ref, acc_ref):
    b = pl.program_id(0)
    wio = lax.broadcasted_iota(jnp.int32, (HW, 1), 0) % W

    @pl.when(b == 0)
    def _():
        pad_ref[...] = jnp.zeros((ROWS, CC), F32)

    for ch in range(NCH):
        cs = ch * CC

        def put(img):
            pad_ref[pl.ds(PAD, HW), :] = img

        def conv(wt_ref, kh, kw, ph, pw):
            acc = jnp.zeros((HW, CC), F32)
            t = 0
            for i in range(kh):
                for j in range(kw):
                    delta = (i - ph) * W + (j - pw)
                    sl = pad_ref[pl.ds(PAD + delta, HW), :]
                    dw = j - pw
                    if dw != 0:
                        m = (wio + dw >= 0) & (wio + dw < W)
                        sl = jnp.where(m, sl, 0.0)
                    acc = acc + sl * wt_ref[t:t + 1, cs:cs + CC]
                    t += 1
            return acc

        def brow(i):
            return cb_ref[i:i + 1, cs:cs + CC]

        put(x_ref[0][:, cs:cs + CC])
        s0 = conv(w0, 5, 5, 2, 2) + brow(0)
        s0_ref[...] = s0
        put(s0)
        t0_ref[...] = conv(w01, 1, 7, 0, 3) + brow(1)
        t1_ref[...] = conv(w11, 1, 11, 0, 5) + brow(3)
        aw2 = conv(w21, 1, 21, 0, 10) + brow(5)
        put(aw2)
        acc = s0_ref[...] + conv(w22, 21, 1, 10, 0) + brow(6)
        put(t0_ref[...])
        acc = acc + conv(w02, 7, 1, 3, 0) + brow(2)
        put(t1_ref[...])
        acc = acc + conv(w12, 11, 1, 5, 0) + brow(4)
        acc_ref[:, cs:cs + CC] = acc

    X = x_ref[0]
    attn3 = _dot(acc_ref[...].astype(BF16), W3_ref[...], ((1,), (1,))) \
        + cb_ref[7:8, :]
    xs_ref[0] = attn3 * X
    xh = jnp.zeros((W, C), F32)
    for t in range(H):
        xh = xh + X[t * W:(t + 1) * W, :]
    xh_ref[0] = xh


def _stage1(x, taps, cbias, W3):
    w0, w01, w02, w11, w12, w21, w22 = taps
    full = lambda s: pl.BlockSpec(s, lambda b: tuple(0 for _ in s))
    return pl.pallas_call(
        _s1_body,
        grid=(B,),
        in_specs=[pl.BlockSpec((1, HW, C), lambda b: (b, 0, 0))]
        + [full(w.shape) for w in (w0, w01, w02, w11, w12, w21, w22)]
        + [full(cbias.shape), full(W3.shape)],
        out_specs=[pl.BlockSpec((1, HW, C), lambda b: (b, 0, 0)),
                   pl.BlockSpec((1, W, C), lambda b: (b, 0, 0))],
        out_shape=[jax.ShapeDtypeStruct((B, HW, C), F32),
                   jax.ShapeDtypeStruct((B, W, C), F32)],
        scratch_shapes=[pltpu.VMEM((ROWS, CC), F32), pltpu.VMEM((HW, CC), F32),
                        pltpu.VMEM((HW, CC), F32), pltpu.VMEM((HW, CC), F32),
                        pltpu.VMEM((HW, C), F32)],
        compiler_params=pltpu.CompilerParams(
            dimension_semantics=("arbitrary",)),
    )(x, w0, w01, w02, w11, w12, w21, w22, cbias, W3)


def _s2_body(xs_ref, wcat_ref, cb_ref, cg_ref, cbe_ref, out_ref, ab_ref):
    b = pl.program_id(0)

    @pl.when(b == 0)
    def _():
        ssum = jnp.zeros((1, C), F32)
        for bb in range(B):
            ssum = ssum + jnp.sum(xs_ref[bb], axis=0, keepdims=True)
        s = ssum / N
        G = jnp.zeros((C, C), F32)
        for bb in range(B):
            xc = (xs_ref[bb] - s).astype(BF16)
            G = G + _dot(xc, xc, ((0,), (0,)))
        G16 = G.astype(BF16)
        s16 = s.astype(BF16)
        ones = jnp.ones((1, C), F32)
        for k in range(K):
            Ws = wcat_ref[k * C:(k + 1) * C, :]
            Ez = _dot(s16, Ws, ((1,), (1,)))
            WG = _dot(Ws, G16, ((1,), (0,)))
            var = _dot(ones, WG * Ws.astype(F32), ((1,), (1,))) / N
            alpha = cg_ref[:, k * C:(k + 1) * C] / jnp.sqrt(var + 1e-5)
            beta = cbe_ref[:, k * C:(k + 1) * C] - Ez * alpha
            ab_ref[pl.ds(0, 1), k * C:(k + 1) * C] = alpha
            ab_ref[pl.ds(1, 1), k * C:(k + 1) * C] = beta

    Z = _dot(xs_ref[b].astype(BF16), wcat_ref[...], ((1,), (1,)))
    y = jnp.maximum(Z * ab_ref[pl.ds(0, 1), :] + ab_ref[pl.ds(1, 1), :], 0.0)
    ones_n = jnp.ones((1, HW), F32)
    out_ref[...] = _dot(ones_n, y, ((1,), (0,)))[None] / HW


def _stage2(xs, cW, cb, cg, cbe):
    wcat = cW.reshape(K * C, C).astype(BF16)
    cbf = cb.reshape(1, K * C)
    cgf = cg.reshape(1, K * C)
    cbef = cbe.reshape(1, K * C)
    full = lambda s: pl.BlockSpec(s, lambda b: tuple(0 for _ in s))
    out = pl.pallas_call(
        _s2_body,
        grid=(B,),
        in_specs=[full((B, HW, C)), full((K * C, C)), full((1, K * C)),
                  full((1, K * C)), full((1, K * C))],
        out_specs=pl.BlockSpec((1, 1, K * C), lambda b: (b, 0, 0)),
        out_shape=jax.ShapeDtypeStruct((B, 1, K * C), F32),
        scratch_shapes=[pltpu.VMEM((2, K * C), F32)],
        compiler_params=pltpu.CompilerParams(
            dimension_semantics=("arbitrary",)),
    )(xs, wcat, cbf, cgf, cbef)
    return out.reshape(B, K, C)


def _s3_body(v_ref, xh_ref, VW_ref, Vb_ref, UW_ref, Ub_ref, gK_ref, bK_ref,
             c1w_ref, c1b_ref, c2w_ref, c2b_ref, c3w_ref, c3b_ref,
             c4w_ref, c4b_ref, cclW_ref, cclb_ref, cclg_ref, cclbe_ref,
             pW_ref, pb_ref, pg_ref, pbe_ref, sc_ref,
             out_ref, agg_ref, fp_ref):
    ones_row = jnp.ones((1, C), F32)
    pL = pW_ref[:, 0:C]
    pR = pW_ref[:, C:2 * C]

    pool_rows = []
    for b in range(B):
        XHb = xh_ref[b]
        XHm = XHb * (1.0 / W)
        x1 = _dot(XHm, c1w_ref[...], ((1,), (1,))) + c1b_ref[...]
        x2 = _dot(XHm, c2w_ref[...], ((1,), (1,))) + c2b_ref[...]
        TT = jnp.zeros((W, REL), F32)
        for u in range(W):
            TT = TT + jnp.tanh(x1[u:u + 1, :] - x2)
        P1T = _dot(TT, c4w_ref[...], ((1,), (1,))) + float(W) * c4b_ref[...]
        S3T = _dot(XHb, c3w_ref[...], ((1,), (1,))) + float(W) * c3b_ref[...]
        pool_rows.append(jnp.sum(P1T * S3T, axis=0, keepdims=True) / float(HW))
    pool = jnp.concatenate(pool_rows, axis=0)

    y1 = _dot(pool, cclW_ref[...], ((1,), (1,))) + cclb_ref[...]
    m2 = jnp.mean(y1, axis=0, keepdims=True)
    v2 = jnp.mean((y1 - m2) ** 2, axis=0, keepdims=True)
    lin = jnp.maximum(
        (y1 - m2) * lax.rsqrt(v2 + 1e-5) * cclg_ref[...] + cclbe_ref[...], 0.0)

    msum = jnp.zeros((K, 1), F32)
    iot = lax.broadcasted_iota(jnp.int32, (K, K), 1)
    for b in range(B):
        vb = v_ref[b]
        si = _dot(vb, vb, ((1,), (1,)))
        cur = si
        thr = si
        for _ in range(4):
            m = jnp.max(cur, axis=1, keepdims=True)
            thr = m
            eq = cur == m
            idx = jnp.min(jnp.where(eq, iot, 10 ** 6), axis=1, keepdims=True)
            cur = jnp.where(iot == idx, -1e30, cur)
        adj = (si >= thr).astype(F32)
        deg = jnp.sum(adj, axis=1, keepdims=True)
        dinv = lax.rsqrt(deg)
        Vx = _dot(vb, VW_ref[...], ((1,), (1,))) + Vb_ref[...]
        Ux = _dot(vb, UW_ref[...], ((1,), (1,))) + Ub_ref[...]
        aggb = dinv * _dot(adj, dinv * Vx, ((1,), (0,))) + Ux
        agg_ref[b] = aggb
        msum = msum + jnp.sum(aggb, axis=1, keepdims=True)
    mcol = msum / float(B * C)

    vsum = jnp.zeros((K, 1), F32)
    for b in range(B):
        d = agg_ref[b] - mcol
        vsum = vsum + jnp.sum(d * d, axis=1, keepdims=True)
    inv_col = lax.rsqrt(vsum / float(B * C) + 1e-5)

    ms3 = jnp.zeros((1, C), F32)
    for b in range(B):
        aggn = (agg_ref[b] - mcol) * inv_col * gK_ref[...] + bK_ref[...]
        vgnn = jnp.maximum(v_ref[b] + aggn, 0.0)
        fpb = (_dot(vgnn, pL, ((1,), (1,)))
               + _dot(lin[b:b + 1, :], pR, ((1,), (1,)))
               + pb_ref[...])
        fp_ref[b] = fpb
        ms3 = ms3 + jnp.sum(fpb, axis=0, keepdims=True)
    m3 = ms3 / float(B * K)

    vs3 = jnp.zeros((1, C), F32)
    for b in range(B):
        d = fp_ref[b] - m3
        vs3 = vs3 + jnp.sum(d * d, axis=0, keepdims=True)
    inv3 = lax.rsqrt(vs3 / float(B * K) + 1e-5)

    scr = jnp.maximum(sc_ref[...], 0.0)
    sn = jnp.sqrt(jnp.sum(scr * scr, axis=1, keepdims=True))
    scn = scr / jnp.maximum(sn, 1e-12)

    rows = []
    for b in range(B):
        y = (fp_ref[b] - m3) * inv3 * pg_ref[...] + pbe_ref[...]
        f = jnp.maximum(y, 0.0)
        fn = jnp.sqrt(jnp.sum(f * f, axis=1, keepdims=True))
        fhat = f / jnp.maximum(fn, 1e-12)
        rows.append(_dot(ones_row, fhat * scn, ((1,), (1,))))
    out_ref[...] = jnp.concatenate(rows, axis=0)


def _stage3(v_bkc, XH, args):
    full = lambda a: pl.BlockSpec(a.shape, lambda: tuple(0 for _ in a.shape))
    ins = [v_bkc, XH] + list(args)
    return pl.pallas_call(
        _s3_body,
        in_specs=[full(a) for a in ins],
        out_specs=pl.BlockSpec((B, K), lambda: (0, 0)),
        out_shape=jax.ShapeDtypeStruct((B, K), F32),
        scratch_shapes=[pltpu.VMEM((B, K, C), F32), pltpu.VMEM((B, K, C), F32)],
    )(*ins)


def kernel(x, params):
    p = params

    def taps(name, kh, kw):
        w = p[name]["w"]
        return w[:, 0].reshape(C, kh * kw).T

    tap_ws = (taps("conv0", 5, 5), taps("conv0_1", 1, 7), taps("conv0_2", 7, 1),
              taps("conv1_1", 1, 11), taps("conv1_2", 11, 1),
              taps("conv2_1", 1, 21), taps("conv2_2", 21, 1))
    cbias = jnp.stack([p["conv0"]["b"], p["conv0_1"]["b"], p["conv0_2"]["b"],
                       p["conv1_1"]["b"], p["conv1_2"]["b"],
                       p["conv2_1"]["b"], p["conv2_2"]["b"],
                       p["conv3"]["b"]], axis=0)
    W3 = p["conv3"]["w"][:, :, 0, 0].astype(BF16)

    xs, XH = _stage1(x, tap_ws, cbias, W3)

    v_bkc = _stage2(xs, p["cls_W"], p["cls_b"], p["cls_g"], p["cls_be"])

    row = lambda a: a[None, :]
    args = (p["V_W"], row(p["V_b"]), p["U_W"], row(p["U_b"]),
            jnp.broadcast_to(p["bnv_g"][:, None], (K, C)),
            jnp.broadcast_to(p["bnv_b"][:, None], (K, C)),
            p["c1_w"][:, :, 0, 0], row(p["c1_b"]),
            p["c2_w"][:, :, 0, 0], row(p["c2_b"]),
            p["c3_w"][:, :, 0, 0], row(p["c3_b"]),
            p["c4_w"][:, :, 0, 0], row(p["c4_b"]),
            p["ccl_W"], row(p["ccl_b"]), row(p["ccl_g"]), row(p["ccl_be"]),
            p["post_W"], row(p["post_b"]),
            row(p["post_g"]), row(p["post_be"]), p["sc"])
    return _stage3(v_bkc, XH, args)


# speedup vs baseline: 1.3582x; 1.0495x over previous
import jax
import jax.numpy as jnp
from jax import lax
from jax.experimental import pallas as pl
from jax.experimental.pallas import tpu as pltpu

B, HW, C, K = 16, 196, 384, 12
REL = 48
H = W = 14
N = B * HW
PAD = 144
ROWS = PAD + HW + PAD
F32 = jnp.float32
BF16 = jnp.bfloat16


def _dot(a, b, dims):
    return lax.dot_general(a, b, (dims, ((), ())), preferred_element_type=F32)


CC = 128
NCH = C // CC


def _s1_body(x_ref, w0, w01, w02, w11, w12, w21, w22, cb_ref, W3_ref,
             xs_ref, xh_ref, pad_ref, s0_ref, t0_ref, t1_ref, acc_ref):
    b = pl.program_id(0)
    wio = lax.broadcasted_iota(jnp.int32, (HW, 1), 0) % W
    masks = {dw: (wio + dw >= 0) & (wio + dw < W)
             for dw in range(-10, 11) if dw != 0}

    @pl.when(b == 0)
    def _():
        pad_ref[...] = jnp.zeros((ROWS, CC), F32)

    for ch in range(NCH):
        cs = ch * CC

        def put(img):
            pad_ref[pl.ds(PAD, HW), :] = img

        def conv(wt_ref, kh, kw, ph, pw):
            acc = jnp.zeros((HW, CC), F32)
            t = 0
            for i in range(kh):
                for j in range(kw):
                    delta = (i - ph) * W + (j - pw)
                    sl = pad_ref[pl.ds(PAD + delta, HW), :]
                    dw = j - pw
                    if dw != 0:
                        sl = jnp.where(masks[dw], sl, 0.0)
                    acc = acc + sl * wt_ref[t:t + 1, cs:cs + CC]
                    t += 1
            return acc

        def brow(i):
            return cb_ref[i:i + 1, cs:cs + CC]

        put(x_ref[0][:, cs:cs + CC])
        s0 = conv(w0, 5, 5, 2, 2) + brow(0)
        s0_ref[...] = s0
        put(s0)
        t0_ref[...] = conv(w01, 1, 7, 0, 3) + brow(1)
        t1_ref[...] = conv(w11, 1, 11, 0, 5) + brow(3)
        aw2 = conv(w21, 1, 21, 0, 10) + brow(5)
        put(aw2)
        acc = s0_ref[...] + conv(w22, 21, 1, 10, 0) + brow(6)
        put(t0_ref[...])
        acc = acc + conv(w02, 7, 1, 3, 0) + brow(2)
        put(t1_ref[...])
        acc = acc + conv(w12, 11, 1, 5, 0) + brow(4)
        acc_ref[:, cs:cs + CC] = acc

    X = x_ref[0]
    attn3 = _dot(acc_ref[...].astype(BF16), W3_ref[...], ((1,), (1,))) \
        + cb_ref[7:8, :]
    xs_ref[0] = attn3 * X
    xh = jnp.zeros((W, C), F32)
    for t in range(H):
        xh = xh + X[t * W:(t + 1) * W, :]
    xh_ref[0] = xh


def _stage1(x, taps, cbias, W3):
    w0, w01, w02, w11, w12, w21, w22 = taps
    full = lambda s: pl.BlockSpec(s, lambda b: tuple(0 for _ in s))
    return pl.pallas_call(
        _s1_body,
        grid=(B,),
        in_specs=[pl.BlockSpec((1, HW, C), lambda b: (b, 0, 0))]
        + [full(w.shape) for w in (w0, w01, w02, w11, w12, w21, w22)]
        + [full(cbias.shape), full(W3.shape)],
        out_specs=[pl.BlockSpec((1, HW, C), lambda b: (b, 0, 0)),
                   pl.BlockSpec((1, W, C), lambda b: (b, 0, 0))],
        out_shape=[jax.ShapeDtypeStruct((B, HW, C), F32),
                   jax.ShapeDtypeStruct((B, W, C), F32)],
        scratch_shapes=[pltpu.VMEM((ROWS, CC), F32), pltpu.VMEM((HW, CC), F32),
                        pltpu.VMEM((HW, CC), F32), pltpu.VMEM((HW, CC), F32),
                        pltpu.VMEM((HW, C), F32)],
        compiler_params=pltpu.CompilerParams(
            dimension_semantics=("arbitrary",)),
    )(x, w0, w01, w02, w11, w12, w21, w22, cbias, W3)


def _s2_body(xs_ref, wcat_ref, cb_ref, cg_ref, cbe_ref, out_ref, ab_ref):
    b = pl.program_id(0)

    @pl.when(b == 0)
    def _():
        ssum = jnp.zeros((1, C), F32)
        for bb in range(B):
            ssum = ssum + jnp.sum(xs_ref[bb], axis=0, keepdims=True)
        s = ssum / N
        G = jnp.zeros((C, C), F32)
        for bb in range(B):
            xc = (xs_ref[bb] - s).astype(BF16)
            G = G + _dot(xc, xc, ((0,), (0,)))
        G16 = G.astype(BF16)
        s16 = s.astype(BF16)
        ones = jnp.ones((1, C), F32)
        for k in range(K):
            Ws = wcat_ref[k * C:(k + 1) * C, :]
            Ez = _dot(s16, Ws, ((1,), (1,)))
            WG = _dot(Ws, G16, ((1,), (0,)))
            var = _dot(ones, WG * Ws.astype(F32), ((1,), (1,))) / N
            alpha = cg_ref[:, k * C:(k + 1) * C] / jnp.sqrt(var + 1e-5)
            beta = cbe_ref[:, k * C:(k + 1) * C] - Ez * alpha
            ab_ref[pl.ds(0, 1), k * C:(k + 1) * C] = alpha
            ab_ref[pl.ds(1, 1), k * C:(k + 1) * C] = beta

    Z = _dot(xs_ref[b].astype(BF16), wcat_ref[...], ((1,), (1,)))
    y = jnp.maximum(Z * ab_ref[pl.ds(0, 1), :] + ab_ref[pl.ds(1, 1), :], 0.0)
    ones_n = jnp.ones((1, HW), F32)
    out_ref[...] = _dot(ones_n, y, ((1,), (0,)))[None] / HW


def _stage2(xs, cW, cb, cg, cbe):
    wcat = cW.reshape(K * C, C).astype(BF16)
    cbf = cb.reshape(1, K * C)
    cgf = cg.reshape(1, K * C)
    cbef = cbe.reshape(1, K * C)
    full = lambda s: pl.BlockSpec(s, lambda b: tuple(0 for _ in s))
    out = pl.pallas_call(
        _s2_body,
        grid=(B,),
        in_specs=[full((B, HW, C)), full((K * C, C)), full((1, K * C)),
                  full((1, K * C)), full((1, K * C))],
        out_specs=pl.BlockSpec((1, 1, K * C), lambda b: (b, 0, 0)),
        out_shape=jax.ShapeDtypeStruct((B, 1, K * C), F32),
        scratch_shapes=[pltpu.VMEM((2, K * C), F32)],
        compiler_params=pltpu.CompilerParams(
            dimension_semantics=("arbitrary",)),
    )(xs, wcat, cbf, cgf, cbef)
    return out.reshape(B, K, C)


BK = B * K
BW = B * W


def _s3_body(v_ref, xh_ref, VW_ref, Vb_ref, UW_ref, Ub_ref, gcol_ref,
             bcol_ref, c1w_ref, c1b_ref, c2w_ref, c2b_ref, c3w_ref, c3b_ref,
             c4w_ref, c4b_ref, cclW_ref, cclb_ref, cclg_ref, cclbe_ref,
             pL_ref, pR_ref, pb_ref, pg_ref, pbe_ref, sc_ref,
             P_ref, Pt_ref, blk_ref, E14_ref, Sall_ref, S14_ref, E16_ref,
             out_ref):
    Vf = v_ref[...]
    Vf16 = Vf.astype(BF16)

    XHf = xh_ref[...]
    XHm16 = (XHf * (1.0 / W)).astype(BF16)
    x1 = _dot(XHm16, c1w_ref[...], ((1,), (1,))) + c1b_ref[...]
    x2 = _dot(XHm16, c2w_ref[...], ((1,), (1,))) + c2b_ref[...]
    S3T = _dot(XHf.astype(BF16), c3w_ref[...], ((1,), (1,))) \
        + float(W) * c3b_ref[...]
    x1sel = _dot(Sall_ref[...], x1.astype(BF16), ((1,), (0,)))
    TT = jnp.zeros((BW, REL), F32)
    for u in range(W):
        xu = x1sel[u * B:(u + 1) * B, :]
        rep = _dot(E14_ref[...], xu, ((1,), (0,)))
        TT = TT + jnp.tanh(rep - x2)
    P1T = _dot(TT.astype(BF16), c4w_ref[...], ((1,), (1,))) \
        + float(W) * c4b_ref[...]
    pool = _dot(S14_ref[...], P1T * S3T, ((1,), (0,))) / float(HW)

    y1 = _dot(pool, cclW_ref[...], ((1,), (1,))) + cclb_ref[...]
    m2 = jnp.mean(y1, axis=0, keepdims=True)
    v2 = jnp.mean((y1 - m2) ** 2, axis=0, keepdims=True)
    lin = jnp.maximum(
        (y1 - m2) * lax.rsqrt(v2 + 1e-5) * cclg_ref[...] + cclbe_ref[...], 0.0)
    linfull = _dot(E16_ref[...], lin, ((1,), (0,)))

    S = _dot(Vf, Vf, ((1,), (1,)))
    blk = blk_ref[...] > 0.5
    cur = jnp.where(blk, S, -1e30)
    thr = cur
    iot = lax.broadcasted_iota(jnp.int32, (BK, BK), 1)
    for _ in range(4):
        m = jnp.max(cur, axis=1, keepdims=True)
        thr = m
        eq = cur == m
        idx = jnp.min(jnp.where(eq, iot, 10 ** 6), axis=1, keepdims=True)
        cur = jnp.where(iot == idx, -1e30, cur)
    adj = jnp.where((S >= thr) & blk, 1.0, 0.0)
    deg = jnp.sum(adj, axis=1, keepdims=True)
    dinv = lax.rsqrt(deg)

    Vx = _dot(Vf16, VW_ref[...], ((1,), (1,))) + Vb_ref[...]
    Ux = _dot(Vf16, UW_ref[...], ((1,), (1,))) + Ub_ref[...]
    agg = dinv * _dot(adj.astype(BF16), (dinv * Vx).astype(BF16),
                      ((1,), (0,))) + Ux

    msum = jnp.sum(_dot(P_ref[...], agg, ((1,), (0,))), axis=1,
                   keepdims=True)
    mexp = _dot(Pt_ref[...], msum / float(B * C), ((1,), (0,)))
    d = agg - mexp
    vsum = jnp.sum(_dot(P_ref[...], d * d, ((1,), (0,))), axis=1,
                   keepdims=True)
    invc = lax.rsqrt(vsum / float(B * C) + 1e-5)
    invexp = _dot(Pt_ref[...], invc, ((1,), (0,)))
    vgnn = jnp.maximum(Vf + d * invexp * gcol_ref[...] + bcol_ref[...], 0.0)

    fp = (_dot(vgnn.astype(BF16), pL_ref[...], ((1,), (1,)))
          + _dot(linfull.astype(BF16), pR_ref[...], ((1,), (1,)))
          + pb_ref[...])
    ones1 = jnp.ones((1, BK), F32)
    m3 = _dot(ones1, fp, ((1,), (0,))) / float(BK)
    d3 = fp - m3
    v3 = _dot(ones1, d3 * d3, ((1,), (0,))) / float(BK)
    f = jnp.maximum(d3 * lax.rsqrt(v3 + 1e-5) * pg_ref[...] + pbe_ref[...],
                    0.0)
    fn = jnp.sqrt(jnp.sum(f * f, axis=1, keepdims=True))
    fhat = f / jnp.maximum(fn, 1e-12)

    scr = jnp.maximum(sc_ref[...], 0.0)
    sn = jnp.sqrt(jnp.sum(scr * scr, axis=1, keepdims=True))
    scn = scr / jnp.maximum(sn, 1e-12)
    scn_full = _dot(Pt_ref[...], scn, ((1,), (0,)))
    out_ref[...] = jnp.sum(fhat * scn_full, axis=1, keepdims=True)


def _stage3(v_bkc, XH, p):
    row = lambda a: a[None, :]
    v2 = v_bkc.reshape(BK, C)
    xh2 = XH.reshape(BW, C)
    ik = jnp.arange(BK)
    iw = jnp.arange(BW)
    P = (ik[None, :] % K == jnp.arange(K)[:, None]).astype(F32)
    Pt = P.T
    blkm = (ik[:, None] // K == ik[None, :] // K).astype(F32)
    E14 = (iw[:, None] // W == jnp.arange(B)[None, :]).astype(F32)
    Sall = (iw[None, :] == ((iw % B) * W + iw // B)[:, None]).astype(BF16)
    S14 = E14.T
    E16 = (ik[:, None] // K == jnp.arange(B)[None, :]).astype(F32)
    args = (v2, xh2,
            p["V_W"].astype(BF16), row(p["V_b"]),
            p["U_W"].astype(BF16), row(p["U_b"]),
            jnp.tile(p["bnv_g"], B).reshape(BK, 1),
            jnp.tile(p["bnv_b"], B).reshape(BK, 1),
            p["c1_w"][:, :, 0, 0].astype(BF16), row(p["c1_b"]),
            p["c2_w"][:, :, 0, 0].astype(BF16), row(p["c2_b"]),
            p["c3_w"][:, :, 0, 0].astype(BF16), row(p["c3_b"]),
            p["c4_w"][:, :, 0, 0].astype(BF16), row(p["c4_b"]),
            p["ccl_W"], row(p["ccl_b"]), row(p["ccl_g"]), row(p["ccl_be"]),
            p["post_W"][:, 0:C].astype(BF16), p["post_W"][:, C:2 * C].astype(BF16),
            row(p["post_b"]), row(p["post_g"]), row(p["post_be"]), p["sc"],
            P, Pt, blkm, E14, Sall, S14, E16)
    full = lambda a: pl.BlockSpec(a.shape, lambda: tuple(0 for _ in a.shape))
    out = pl.pallas_call(
        _s3_body,
        in_specs=[full(a) for a in args],
        out_specs=pl.BlockSpec((BK, 1), lambda: (0, 0)),
        out_shape=jax.ShapeDtypeStruct((BK, 1), F32),
    )(*args)
    return out.reshape(B, K)


def kernel(x, params):
    p = params

    def taps(name, kh, kw):
        w = p[name]["w"]
        return w[:, 0].reshape(C, kh * kw).T

    tap_ws = (taps("conv0", 5, 5), taps("conv0_1", 1, 7), taps("conv0_2", 7, 1),
              taps("conv1_1", 1, 11), taps("conv1_2", 11, 1),
              taps("conv2_1", 1, 21), taps("conv2_2", 21, 1))
    cbias = jnp.stack([p["conv0"]["b"], p["conv0_1"]["b"], p["conv0_2"]["b"],
                       p["conv1_1"]["b"], p["conv1_2"]["b"],
                       p["conv2_1"]["b"], p["conv2_2"]["b"],
                       p["conv3"]["b"]], axis=0)
    W3 = p["conv3"]["w"][:, :, 0, 0].astype(BF16)

    xs, XH = _stage1(x, tap_ws, cbias, W3)

    v_bkc = _stage2(xs, p["cls_W"], p["cls_b"], p["cls_g"], p["cls_be"])

    return _stage3(v_bkc, XH, p)


# speedup vs baseline: 1.4233x; 1.0480x over previous
import jax
import jax.numpy as jnp
from jax import lax
from jax.experimental import pallas as pl
from jax.experimental.pallas import tpu as pltpu

B, HW, C, K = 16, 196, 384, 12
REL = 48
H = W = 14
N = B * HW
PAD = 144
ROWS = PAD + HW + PAD
F32 = jnp.float32
BF16 = jnp.bfloat16


def _dot(a, b, dims):
    return lax.dot_general(a, b, (dims, ((), ())), preferred_element_type=F32)


CC = 128
NCH = C // CC
CONV_GEOM = ((5, 5, 2, 2), (1, 7, 0, 3), (7, 1, 3, 0), (1, 11, 0, 5),
             (11, 1, 5, 0), (1, 21, 0, 10), (21, 1, 10, 0))
_BASES = []
_mi = 0
for _kh, _kw, _ph, _pw in CONV_GEOM:
    _BASES.append(_mi)
    _mi += sum(1 for _j in range(_kw) if _j - _pw != 0) * _kh
MT = _mi


def _s1_body(x_ref, w0, w01, w02, w11, w12, w21, w22, cb_ref, W3_ref,
             xs_ref, xh_ref, pad_ref, s0_ref, t0_ref, t1_ref, acc_ref,
             mw_ref):
    b = pl.program_id(0)
    wrefs = (w0, w01, w02, w11, w12, w21, w22)

    @pl.when(b == 0)
    def _():
        pad_ref[...] = jnp.zeros((ROWS, CC), F32)
        wio = lax.broadcasted_iota(jnp.int32, (HW, 1), 0) % W
        for ci, (kh, kw, ph, pw) in enumerate(CONV_GEOM):
            k = _BASES[ci]
            t = 0
            for i in range(kh):
                for j in range(kw):
                    dw = j - pw
                    if dw != 0:
                        m = (wio + dw >= 0) & (wio + dw < W)
                        wrow = wrefs[ci][t:t + 1, :]
                        mw_ref[pl.ds(k * HW, HW), :] = jnp.where(
                            m, jnp.broadcast_to(wrow, (HW, C)), 0.0)
                        k += 1
                    t += 1

    for ch in range(NCH):
        cs = ch * CC

        def put(img):
            pad_ref[pl.ds(PAD, HW), :] = img

        def conv(ci):
            kh, kw, ph, pw = CONV_GEOM[ci]
            wt_ref = wrefs[ci]
            acc = jnp.zeros((HW, CC), F32)
            t = 0
            k = _BASES[ci]
            for i in range(kh):
                for j in range(kw):
                    delta = (i - ph) * W + (j - pw)
                    sl = pad_ref[pl.ds(PAD + delta, HW), :]
                    if j - pw != 0:
                        acc = acc + sl * mw_ref[pl.ds(k * HW, HW),
                                                cs:cs + CC]
                        k += 1
                    else:
                        acc = acc + sl * wt_ref[t:t + 1, cs:cs + CC]
                    t += 1
            return acc

        def brow(i):
            return cb_ref[i:i + 1, cs:cs + CC]

        put(x_ref[0][:, cs:cs + CC])
        s0 = conv(0) + brow(0)
        s0_ref[...] = s0
        put(s0)
        t0_ref[...] = conv(1) + brow(1)
        t1_ref[...] = conv(3) + brow(3)
        aw2 = conv(5) + brow(5)
        put(aw2)
        acc = s0_ref[...] + conv(6) + brow(6)
        put(t0_ref[...])
        acc = acc + conv(2) + brow(2)
        put(t1_ref[...])
        acc = acc + conv(4) + brow(4)
        acc_ref[:, cs:cs + CC] = acc

    X = x_ref[0]
    attn3 = _dot(acc_ref[...].astype(BF16), W3_ref[...], ((1,), (1,))) \
        + cb_ref[7:8, :]
    xs_ref[0] = attn3 * X
    xh = jnp.zeros((W, C), F32)
    for t in range(H):
        xh = xh + X[t * W:(t + 1) * W, :]
    xh_ref[0] = xh


def _stage1(x, taps, cbias, W3):
    w0, w01, w02, w11, w12, w21, w22 = taps
    full = lambda s: pl.BlockSpec(s, lambda b: tuple(0 for _ in s))
    return pl.pallas_call(
        _s1_body,
        grid=(B,),
        in_specs=[pl.BlockSpec((1, HW, C), lambda b: (b, 0, 0))]
        + [full(w.shape) for w in (w0, w01, w02, w11, w12, w21, w22)]
        + [full(cbias.shape), full(W3.shape)],
        out_specs=[pl.BlockSpec((1, HW, C), lambda b: (b, 0, 0)),
                   pl.BlockSpec((1, W, C), lambda b: (b, 0, 0))],
        out_shape=[jax.ShapeDtypeStruct((B, HW, C), F32),
                   jax.ShapeDtypeStruct((B, W, C), F32)],
        scratch_shapes=[pltpu.VMEM((ROWS, CC), F32), pltpu.VMEM((HW, CC), F32),
                        pltpu.VMEM((HW, CC), F32), pltpu.VMEM((HW, CC), F32),
                        pltpu.VMEM((HW, C), F32),
                        pltpu.VMEM((MT * HW, C), F32)],
        compiler_params=pltpu.CompilerParams(
            dimension_semantics=("arbitrary",)),
    )(x, w0, w01, w02, w11, w12, w21, w22, cbias, W3)


def _s2_body(xs_ref, wcat_ref, cb_ref, cg_ref, cbe_ref, out_ref, ab_ref):
    b = pl.program_id(0)

    @pl.when(b == 0)
    def _():
        ssum = jnp.zeros((1, C), F32)
        for bb in range(B):
            ssum = ssum + jnp.sum(xs_ref[bb], axis=0, keepdims=True)
        s = ssum / N
        G = jnp.zeros((C, C), F32)
        for bb in range(B):
            xc = (xs_ref[bb] - s).astype(BF16)
            G = G + _dot(xc, xc, ((0,), (0,)))
        G16 = G.astype(BF16)
        s16 = s.astype(BF16)
        ones = jnp.ones((1, C), F32)
        for k in range(K):
            Ws = wcat_ref[k * C:(k + 1) * C, :]
            Ez = _dot(s16, Ws, ((1,), (1,)))
            WG = _dot(Ws, G16, ((1,), (0,)))
            var = _dot(ones, WG * Ws.astype(F32), ((1,), (1,))) / N
            alpha = cg_ref[:, k * C:(k + 1) * C] / jnp.sqrt(var + 1e-5)
            beta = cbe_ref[:, k * C:(k + 1) * C] - Ez * alpha
            ab_ref[pl.ds(0, 1), k * C:(k + 1) * C] = alpha
            ab_ref[pl.ds(1, 1), k * C:(k + 1) * C] = beta

    Z = _dot(xs_ref[b].astype(BF16), wcat_ref[...], ((1,), (1,)))
    y = jnp.maximum(Z * ab_ref[pl.ds(0, 1), :] + ab_ref[pl.ds(1, 1), :], 0.0)
    ones_n = jnp.ones((1, HW), F32)
    out_ref[...] = _dot(ones_n, y, ((1,), (0,)))[None] / HW


def _stage2(xs, cW, cb, cg, cbe):
    wcat = cW.reshape(K * C, C).astype(BF16)
    cbf = cb.reshape(1, K * C)
    cgf = cg.reshape(1, K * C)
    cbef = cbe.reshape(1, K * C)
    full = lambda s: pl.BlockSpec(s, lambda b: tuple(0 for _ in s))
    out = pl.pallas_call(
        _s2_body,
        grid=(B,),
        in_specs=[full((B, HW, C)), full((K * C, C)), full((1, K * C)),
                  full((1, K * C)), full((1, K * C))],
        out_specs=pl.BlockSpec((1, 1, K * C), lambda b: (b, 0, 0)),
        out_shape=jax.ShapeDtypeStruct((B, 1, K * C), F32),
        scratch_shapes=[pltpu.VMEM((2, K * C), F32)],
        compiler_params=pltpu.CompilerParams(
            dimension_semantics=("arbitrary",)),
    )(xs, wcat, cbf, cgf, cbef)
    return out.reshape(B, K, C)


BK = B * K
BW = B * W


def _s3_body(v_ref, xh_ref, VW_ref, Vb_ref, UW_ref, Ub_ref, gcol_ref,
             bcol_ref, c1w_ref, c1b_ref, c2w_ref, c2b_ref, c3w_ref, c3b_ref,
             c4w_ref, c4b_ref, cclW_ref, cclb_ref, cclg_ref, cclbe_ref,
             pL_ref, pR_ref, pb_ref, pg_ref, pbe_ref, sc_ref,
             P_ref, Pt_ref, blk_ref, E14_ref, Sall_ref, S14_ref, E16_ref,
             out_ref):
    Vf = v_ref[...]
    Vf16 = Vf.astype(BF16)

    XHf = xh_ref[...]
    XHm16 = (XHf * (1.0 / W)).astype(BF16)
    x1 = _dot(XHm16, c1w_ref[...], ((1,), (1,))) + c1b_ref[...]
    x2 = _dot(XHm16, c2w_ref[...], ((1,), (1,))) + c2b_ref[...]
    S3T = _dot(XHf.astype(BF16), c3w_ref[...], ((1,), (1,))) \
        + float(W) * c3b_ref[...]
    x1sel = _dot(Sall_ref[...], x1.astype(BF16), ((1,), (0,)))
    TT = jnp.zeros((BW, REL), F32)
    for u in range(W):
        xu = x1sel[u * B:(u + 1) * B, :]
        rep = _dot(E14_ref[...], xu, ((1,), (0,)))
        TT = TT + jnp.tanh(rep - x2)
    P1T = _dot(TT.astype(BF16), c4w_ref[...], ((1,), (1,))) \
        + float(W) * c4b_ref[...]
    pool = _dot(S14_ref[...], P1T * S3T, ((1,), (0,))) / float(HW)

    y1 = _dot(pool, cclW_ref[...], ((1,), (1,))) + cclb_ref[...]
    m2 = jnp.mean(y1, axis=0, keepdims=True)
    v2 = jnp.mean((y1 - m2) ** 2, axis=0, keepdims=True)
    lin = jnp.maximum(
        (y1 - m2) * lax.rsqrt(v2 + 1e-5) * cclg_ref[...] + cclbe_ref[...], 0.0)
    linfull = _dot(E16_ref[...], lin, ((1,), (0,)))

    S = _dot(Vf, Vf, ((1,), (1,)))
    blk = blk_ref[...] > 0.5
    cur = jnp.where(blk, S, -1e30)
    thr = cur
    iot = lax.broadcasted_iota(jnp.int32, (BK, BK), 1)
    for _ in range(4):
        m = jnp.max(cur, axis=1, keepdims=True)
        thr = m
        eq = cur == m
        idx = jnp.min(jnp.where(eq, iot, 10 ** 6), axis=1, keepdims=True)
        cur = jnp.where(iot == idx, -1e30, cur)
    adj = jnp.where((S >= thr) & blk, 1.0, 0.0)
    deg = jnp.sum(adj, axis=1, keepdims=True)
    dinv = lax.rsqrt(deg)

    Vx = _dot(Vf16, VW_ref[...], ((1,), (1,))) + Vb_ref[...]
    Ux = _dot(Vf16, UW_ref[...], ((1,), (1,))) + Ub_ref[...]
    agg = dinv * _dot(adj.astype(BF16), (dinv * Vx).astype(BF16),
                      ((1,), (0,))) + Ux

    msum = jnp.sum(_dot(P_ref[...], agg, ((1,), (0,))), axis=1,
                   keepdims=True)
    mexp = _dot(Pt_ref[...], msum / float(B * C), ((1,), (0,)))
    d = agg - mexp
    vsum = jnp.sum(_dot(P_ref[...], d * d, ((1,), (0,))), axis=1,
                   keepdims=True)
    invc = lax.rsqrt(vsum / float(B * C) + 1e-5)
    invexp = _dot(Pt_ref[...], invc, ((1,), (0,)))
    vgnn = jnp.maximum(Vf + d * invexp * gcol_ref[...] + bcol_ref[...], 0.0)

    fp = (_dot(vgnn.astype(BF16), pL_ref[...], ((1,), (1,)))
          + _dot(linfull.astype(BF16), pR_ref[...], ((1,), (1,)))
          + pb_ref[...])
    ones1 = jnp.ones((1, BK), F32)
    m3 = _dot(ones1, fp, ((1,), (0,))) / float(BK)
    d3 = fp - m3
    v3 = _dot(ones1, d3 * d3, ((1,), (0,))) / float(BK)
    f = jnp.maximum(d3 * lax.rsqrt(v3 + 1e-5) * pg_ref[...] + pbe_ref[...],
                    0.0)
    fn = jnp.sqrt(jnp.sum(f * f, axis=1, keepdims=True))
    fhat = f / jnp.maximum(fn, 1e-12)

    scr = jnp.maximum(sc_ref[...], 0.0)
    sn = jnp.sqrt(jnp.sum(scr * scr, axis=1, keepdims=True))
    scn = scr / jnp.maximum(sn, 1e-12)
    scn_full = _dot(Pt_ref[...], scn, ((1,), (0,)))
    out_ref[...] = jnp.sum(fhat * scn_full, axis=1, keepdims=True)


def _stage3(v_bkc, XH, p):
    row = lambda a: a[None, :]
    v2 = v_bkc.reshape(BK, C)
    xh2 = XH.reshape(BW, C)
    ik = jnp.arange(BK)
    iw = jnp.arange(BW)
    P = (ik[None, :] % K == jnp.arange(K)[:, None]).astype(F32)
    Pt = P.T
    blkm = (ik[:, None] // K == ik[None, :] // K).astype(F32)
    E14 = (iw[:, None] // W == jnp.arange(B)[None, :]).astype(F32)
    Sall = (iw[None, :] == ((iw % B) * W + iw // B)[:, None]).astype(BF16)
    S14 = E14.T
    E16 = (ik[:, None] // K == jnp.arange(B)[None, :]).astype(F32)
    args = (v2, xh2,
            p["V_W"].astype(BF16), row(p["V_b"]),
            p["U_W"].astype(BF16), row(p["U_b"]),
            jnp.tile(p["bnv_g"], B).reshape(BK, 1),
            jnp.tile(p["bnv_b"], B).reshape(BK, 1),
            p["c1_w"][:, :, 0, 0].astype(BF16), row(p["c1_b"]),
            p["c2_w"][:, :, 0, 0].astype(BF16), row(p["c2_b"]),
            p["c3_w"][:, :, 0, 0].astype(BF16), row(p["c3_b"]),
            p["c4_w"][:, :, 0, 0].astype(BF16), row(p["c4_b"]),
            p["ccl_W"], row(p["ccl_b"]), row(p["ccl_g"]), row(p["ccl_be"]),
            p["post_W"][:, 0:C].astype(BF16), p["post_W"][:, C:2 * C].astype(BF16),
            row(p["post_b"]), row(p["post_g"]), row(p["post_be"]), p["sc"],
            P, Pt, blkm, E14, Sall, S14, E16)
    full = lambda a: pl.BlockSpec(a.shape, lambda: tuple(0 for _ in a.shape))
    out = pl.pallas_call(
        _s3_body,
        in_specs=[full(a) for a in args],
        out_specs=pl.BlockSpec((BK, 1), lambda: (0, 0)),
        out_shape=jax.ShapeDtypeStruct((BK, 1), F32),
    )(*args)
    return out.reshape(B, K)


def kernel(x, params):
    p = params

    def taps(name, kh, kw):
        w = p[name]["w"]
        return w[:, 0].reshape(C, kh * kw).T

    tap_ws = (taps("conv0", 5, 5), taps("conv0_1", 1, 7), taps("conv0_2", 7, 1),
              taps("conv1_1", 1, 11), taps("conv1_2", 11, 1),
              taps("conv2_1", 1, 21), taps("conv2_2", 21, 1))
    cbias = jnp.stack([p["conv0"]["b"], p["conv0_1"]["b"], p["conv0_2"]["b"],
                       p["conv1_1"]["b"], p["conv1_2"]["b"],
                       p["conv2_1"]["b"], p["conv2_2"]["b"],
                       p["conv3"]["b"]], axis=0)
    W3 = p["conv3"]["w"][:, :, 0, 0].astype(BF16)

    xs, XH = _stage1(x, tap_ws, cbias, W3)

    v_bkc = _stage2(xs, p["cls_W"], p["cls_b"], p["cls_g"], p["cls_be"])

    return _stage3(v_bkc, XH, p)


# speedup vs baseline: 1.4399x; 1.0117x over previous
import jax
import jax.numpy as jnp
from jax import lax
from jax.experimental import pallas as pl
from jax.experimental.pallas import tpu as pltpu

B, HW, C, K = 16, 196, 384, 12
REL = 48
H = W = 14
N = B * HW
PAD = 144
ROWS = PAD + HW + PAD
F32 = jnp.float32
BF16 = jnp.bfloat16


def _dot(a, b, dims):
    return lax.dot_general(a, b, (dims, ((), ())), preferred_element_type=F32)


CC = 128
NCH = C // CC
CONV_GEOM = ((5, 5, 2, 2), (1, 7, 0, 3), (7, 1, 3, 0), (1, 11, 0, 5),
             (11, 1, 5, 0), (1, 21, 0, 10), (21, 1, 10, 0))
_BASES = []
_mi = 0
for _kh, _kw, _ph, _pw in CONV_GEOM:
    _BASES.append(_mi)
    _mi += sum(1 for _j in range(_kw) if _j - _pw != 0) * _kh
MT = _mi


def _s1_body(x_ref, w0, w01, w02, w11, w12, w21, w22, cb_ref, W3_ref,
             xs_ref, xh_ref, pad_ref, s0_ref, t0_ref, t1_ref, acc_ref,
             mw_ref):
    b = pl.program_id(0)
    wrefs = (w0, w01, w02, w11, w12, w21, w22)

    @pl.when(b == 0)
    def _():
        pad_ref[...] = jnp.zeros((ROWS, CC), F32)
        wio = lax.broadcasted_iota(jnp.int32, (HW, 1), 0) % W
        for ci, (kh, kw, ph, pw) in enumerate(CONV_GEOM):
            k = _BASES[ci]
            t = 0
            for i in range(kh):
                for j in range(kw):
                    dw = j - pw
                    if dw != 0:
                        m = (wio + dw >= 0) & (wio + dw < W)
                        wrow = wrefs[ci][t:t + 1, :]
                        mw_ref[pl.ds(k * HW, HW), :] = jnp.where(
                            m, jnp.broadcast_to(wrow, (HW, C)), 0.0)
                        k += 1
                    t += 1

    for ch in range(NCH):
        cs = ch * CC

        def put(img):
            pad_ref[pl.ds(PAD, HW), :] = img

        def conv(ci):
            kh, kw, ph, pw = CONV_GEOM[ci]
            wt_ref = wrefs[ci]
            acc = jnp.zeros((HW, CC), F32)
            t = 0
            k = _BASES[ci]
            for i in range(kh):
                for j in range(kw):
                    delta = (i - ph) * W + (j - pw)
                    sl = pad_ref[pl.ds(PAD + delta, HW), :]
                    if j - pw != 0:
                        acc = acc + sl * mw_ref[pl.ds(k * HW, HW),
                                                cs:cs + CC]
                        k += 1
                    else:
                        acc = acc + sl * wt_ref[t:t + 1, cs:cs + CC]
                    t += 1
            return acc

        def brow(i):
            return cb_ref[i:i + 1, cs:cs + CC]

        put(x_ref[0][:, cs:cs + CC])
        s0 = conv(0) + brow(0)
        s0_ref[...] = s0
        put(s0)
        t0_ref[...] = conv(1) + brow(1)
        t1_ref[...] = conv(3) + brow(3)
        aw2 = conv(5) + brow(5)
        put(aw2)
        acc = s0_ref[...] + conv(6) + brow(6)
        put(t0_ref[...])
        acc = acc + conv(2) + brow(2)
        put(t1_ref[...])
        acc = acc + conv(4) + brow(4)
        acc_ref[:, cs:cs + CC] = acc

    X = x_ref[0]
    attn3 = _dot(acc_ref[...].astype(BF16), W3_ref[...], ((1,), (1,))) \
        + cb_ref[7:8, :]
    xs_ref[0] = attn3 * X
    xh = jnp.zeros((W, C), F32)
    for t in range(H):
        xh = xh + X[t * W:(t + 1) * W, :]
    xh_ref[0] = xh


def _stage1(x, taps, cbias, W3):
    w0, w01, w02, w11, w12, w21, w22 = taps
    full = lambda s: pl.BlockSpec(s, lambda b: tuple(0 for _ in s))
    return pl.pallas_call(
        _s1_body,
        grid=(B,),
        in_specs=[pl.BlockSpec((1, HW, C), lambda b: (b, 0, 0))]
        + [full(w.shape) for w in (w0, w01, w02, w11, w12, w21, w22)]
        + [full(cbias.shape), full(W3.shape)],
        out_specs=[pl.BlockSpec((1, HW, C), lambda b: (b, 0, 0)),
                   pl.BlockSpec((1, W, C), lambda b: (b, 0, 0))],
        out_shape=[jax.ShapeDtypeStruct((B, HW, C), F32),
                   jax.ShapeDtypeStruct((B, W, C), F32)],
        scratch_shapes=[pltpu.VMEM((ROWS, CC), F32), pltpu.VMEM((HW, CC), F32),
                        pltpu.VMEM((HW, CC), F32), pltpu.VMEM((HW, CC), F32),
                        pltpu.VMEM((HW, C), F32),
                        pltpu.VMEM((MT * HW, C), F32)],
        compiler_params=pltpu.CompilerParams(
            dimension_semantics=("arbitrary",)),
    )(x, w0, w01, w02, w11, w12, w21, w22, cbias, W3)


def _s2_body(xs_ref, wcat_ref, cb_ref, cg_ref, cbe_ref, out_ref, ab_ref):
    b = pl.program_id(0)

    @pl.when(b == 0)
    def _():
        Xall = xs_ref[...].reshape(N, C)
        onesN = jnp.ones((1, N), F32)
        s = _dot(onesN, Xall, ((1,), (0,))) / N
        Xc16 = (Xall - s).astype(BF16)
        G = _dot(Xc16, Xc16, ((0,), (0,)))
        Ez = _dot(s.astype(BF16), wcat_ref[...], ((1,), (1,)))
        WG = _dot(wcat_ref[...], G.astype(BF16), ((1,), (0,)))
        M = WG * wcat_ref[...].astype(F32)
        ones1 = jnp.ones((1, C), F32)
        var = _dot(ones1, M, ((1,), (1,))) / N
        alpha = cg_ref[...] / jnp.sqrt(var + 1e-5)
        ab_ref[0:1, :] = alpha
        ab_ref[1:2, :] = cbe_ref[...] - Ez * alpha

    Z = _dot(xs_ref[b].astype(BF16), wcat_ref[...], ((1,), (1,)))
    y = jnp.maximum(Z * ab_ref[pl.ds(0, 1), :] + ab_ref[pl.ds(1, 1), :], 0.0)
    ones_n = jnp.ones((1, HW), F32)
    out_ref[...] = _dot(ones_n, y, ((1,), (0,)))[None] / HW


def _stage2(xs, cW, cb, cg, cbe):
    wcat = cW.reshape(K * C, C).astype(BF16)
    cbf = cb.reshape(1, K * C)
    cgf = cg.reshape(1, K * C)
    cbef = cbe.reshape(1, K * C)
    full = lambda s: pl.BlockSpec(s, lambda b: tuple(0 for _ in s))
    out = pl.pallas_call(
        _s2_body,
        grid=(B,),
        in_specs=[full((B, HW, C)), full((K * C, C)), full((1, K * C)),
                  full((1, K * C)), full((1, K * C))],
        out_specs=pl.BlockSpec((1, 1, K * C), lambda b: (b, 0, 0)),
        out_shape=jax.ShapeDtypeStruct((B, 1, K * C), F32),
        scratch_shapes=[pltpu.VMEM((2, K * C), F32)],
        compiler_params=pltpu.CompilerParams(
            dimension_semantics=("arbitrary",)),
    )(xs, wcat, cbf, cgf, cbef)
    return out.reshape(B, K, C)


BK = B * K
BW = B * W


def _s3_body(v_ref, xh_ref, VW_ref, Vb_ref, UW_ref, Ub_ref, gcol_ref,
             bcol_ref, c1w_ref, c1b_ref, c2w_ref, c2b_ref, c3w_ref, c3b_ref,
             c4w_ref, c4b_ref, cclW_ref, cclb_ref, cclg_ref, cclbe_ref,
             pL_ref, pR_ref, pb_ref, pg_ref, pbe_ref, sc_ref,
             P_ref, Pt_ref, blk_ref, E14_ref, Sall_ref, S14_ref, E16_ref,
             out_ref):
    Vf = v_ref[...]
    Vf16 = Vf.astype(BF16)

    XHf = xh_ref[...]
    XHm16 = (XHf * (1.0 / W)).astype(BF16)
    x1 = _dot(XHm16, c1w_ref[...], ((1,), (1,))) + c1b_ref[...]
    x2 = _dot(XHm16, c2w_ref[...], ((1,), (1,))) + c2b_ref[...]
    S3T = _dot(XHf.astype(BF16), c3w_ref[...], ((1,), (1,))) \
        + float(W) * c3b_ref[...]
    x1sel = _dot(Sall_ref[...], x1.astype(BF16), ((1,), (0,)))
    TT = jnp.zeros((BW, REL), F32)
    for u in range(W):
        xu = x1sel[u * B:(u + 1) * B, :]
        rep = _dot(E14_ref[...], xu, ((1,), (0,)))
        TT = TT + jnp.tanh(rep - x2)
    P1T = _dot(TT.astype(BF16), c4w_ref[...], ((1,), (1,))) \
        + float(W) * c4b_ref[...]
    pool = _dot(S14_ref[...], P1T * S3T, ((1,), (0,))) / float(HW)

    y1 = _dot(pool, cclW_ref[...], ((1,), (1,))) + cclb_ref[...]
    m2 = jnp.mean(y1, axis=0, keepdims=True)
    v2 = jnp.mean((y1 - m2) ** 2, axis=0, keepdims=True)
    lin = jnp.maximum(
        (y1 - m2) * lax.rsqrt(v2 + 1e-5) * cclg_ref[...] + cclbe_ref[...], 0.0)
    linfull = _dot(E16_ref[...], lin, ((1,), (0,)))

    S = _dot(Vf, Vf, ((1,), (1,)))
    blk = blk_ref[...] > 0.5
    cur = jnp.where(blk, S, -1e30)
    thr = cur
    iot = lax.broadcasted_iota(jnp.int32, (BK, BK), 1)
    for _ in range(4):
        m = jnp.max(cur, axis=1, keepdims=True)
        thr = m
        eq = cur == m
        idx = jnp.min(jnp.where(eq, iot, 10 ** 6), axis=1, keepdims=True)
        cur = jnp.where(iot == idx, -1e30, cur)
    adj = jnp.where((S >= thr) & blk, 1.0, 0.0)
    deg = jnp.sum(adj, axis=1, keepdims=True)
    dinv = lax.rsqrt(deg)

    Vx = _dot(Vf16, VW_ref[...], ((1,), (1,))) + Vb_ref[...]
    Ux = _dot(Vf16, UW_ref[...], ((1,), (1,))) + Ub_ref[...]
    agg = dinv * _dot(adj.astype(BF16), (dinv * Vx).astype(BF16),
                      ((1,), (0,))) + Ux

    msum = jnp.sum(_dot(P_ref[...], agg, ((1,), (0,))), axis=1,
                   keepdims=True)
    mexp = _dot(Pt_ref[...], msum / float(B * C), ((1,), (0,)))
    d = agg - mexp
    vsum = jnp.sum(_dot(P_ref[...], d * d, ((1,), (0,))), axis=1,
                   keepdims=True)
    invc = lax.rsqrt(vsum / float(B * C) + 1e-5)
    invexp = _dot(Pt_ref[...], invc, ((1,), (0,)))
    vgnn = jnp.maximum(Vf + d * invexp * gcol_ref[...] + bcol_ref[...], 0.0)

    fp = (_dot(vgnn.astype(BF16), pL_ref[...], ((1,), (1,)))
          + _dot(linfull.astype(BF16), pR_ref[...], ((1,), (1,)))
          + pb_ref[...])
    ones1 = jnp.ones((1, BK), F32)
    m3 = _dot(ones1, fp, ((1,), (0,))) / float(BK)
    d3 = fp - m3
    v3 = _dot(ones1, d3 * d3, ((1,), (0,))) / float(BK)
    f = jnp.maximum(d3 * lax.rsqrt(v3 + 1e-5) * pg_ref[...] + pbe_ref[...],
                    0.0)
    fn = jnp.sqrt(jnp.sum(f * f, axis=1, keepdims=True))
    fhat = f / jnp.maximum(fn, 1e-12)

    scr = jnp.maximum(sc_ref[...], 0.0)
    sn = jnp.sqrt(jnp.sum(scr * scr, axis=1, keepdims=True))
    scn = scr / jnp.maximum(sn, 1e-12)
    scn_full = _dot(Pt_ref[...], scn, ((1,), (0,)))
    out_ref[...] = jnp.sum(fhat * scn_full, axis=1, keepdims=True)


def _stage3(v_bkc, XH, p):
    row = lambda a: a[None, :]
    v2 = v_bkc.reshape(BK, C)
    xh2 = XH.reshape(BW, C)
    ik = jnp.arange(BK)
    iw = jnp.arange(BW)
    P = (ik[None, :] % K == jnp.arange(K)[:, None]).astype(F32)
    Pt = P.T
    blkm = (ik[:, None] // K == ik[None, :] // K).astype(F32)
    E14 = (iw[:, None] // W == jnp.arange(B)[None, :]).astype(F32)
    Sall = (iw[None, :] == ((iw % B) * W + iw // B)[:, None]).astype(BF16)
    S14 = E14.T
    E16 = (ik[:, None] // K == jnp.arange(B)[None, :]).astype(F32)
    args = (v2, xh2,
            p["V_W"].astype(BF16), row(p["V_b"]),
            p["U_W"].astype(BF16), row(p["U_b"]),
            jnp.tile(p["bnv_g"], B).reshape(BK, 1),
            jnp.tile(p["bnv_b"], B).reshape(BK, 1),
            p["c1_w"][:, :, 0, 0].astype(BF16), row(p["c1_b"]),
            p["c2_w"][:, :, 0, 0].astype(BF16), row(p["c2_b"]),
            p["c3_w"][:, :, 0, 0].astype(BF16), row(p["c3_b"]),
            p["c4_w"][:, :, 0, 0].astype(BF16), row(p["c4_b"]),
            p["ccl_W"], row(p["ccl_b"]), row(p["ccl_g"]), row(p["ccl_be"]),
            p["post_W"][:, 0:C].astype(BF16), p["post_W"][:, C:2 * C].astype(BF16),
            row(p["post_b"]), row(p["post_g"]), row(p["post_be"]), p["sc"],
            P, Pt, blkm, E14, Sall, S14, E16)
    full = lambda a: pl.BlockSpec(a.shape, lambda: tuple(0 for _ in a.shape))
    out = pl.pallas_call(
        _s3_body,
        in_specs=[full(a) for a in args],
        out_specs=pl.BlockSpec((BK, 1), lambda: (0, 0)),
        out_shape=jax.ShapeDtypeStruct((BK, 1), F32),
    )(*args)
    return out.reshape(B, K)


def kernel(x, params):
    p = params

    def taps(name, kh, kw):
        w = p[name]["w"]
        return w[:, 0].reshape(C, kh * kw).T

    tap_ws = (taps("conv0", 5, 5), taps("conv0_1", 1, 7), taps("conv0_2", 7, 1),
              taps("conv1_1", 1, 11), taps("conv1_2", 11, 1),
              taps("conv2_1", 1, 21), taps("conv2_2", 21, 1))
    cbias = jnp.stack([p["conv0"]["b"], p["conv0_1"]["b"], p["conv0_2"]["b"],
                       p["conv1_1"]["b"], p["conv1_2"]["b"],
                       p["conv2_1"]["b"], p["conv2_2"]["b"],
                       p["conv3"]["b"]], axis=0)
    W3 = p["conv3"]["w"][:, :, 0, 0].astype(BF16)

    xs, XH = _stage1(x, tap_ws, cbias, W3)

    v_bkc = _stage2(xs, p["cls_W"], p["cls_b"], p["cls_g"], p["cls_be"])

    return _stage3(v_bkc, XH, p)
